# baseline (speedup 1.0000x reference)
import jax
import jax.numpy as jnp
from jax import lax
from jax.experimental import pallas as pl
from jax.experimental.pallas import tpu as pltpu
from jax.experimental.pallas import tpu_sc as plsc

VOCAB = 1000000
EMBED_DIM = 64
BATCH = 16384
FIELDS = 26

NC = 2
NS = 16
NW = NC * NS

NBT = BATCH // 128
NTC = FIELDS * NBT
PER_W = NTC // NW
LANE = 128


def _transpose_tile(rows_v, cols_v, dixs):
    @plsc.parallel_loop(0, LANE, unroll=8)
    def _(l):
        ls = jnp.full((16,), l, jnp.int32)
        for g in range(4):
            vec = rows_v[l, pl.ds(g * 16, 16)]
            plsc.store_scatter(cols_v, [dixs[g], ls], vec)


NTC0 = VOCAB // LANE
PER_W0 = NTC0 // NW


def _detr_tile(in_v, op_v, dixs):
    @plsc.parallel_loop(0, LANE, unroll=8)
    def _(l):
        rp = l >> 1
        qb = (l & 1) * EMBED_DIM
        ls = jnp.full((16,), l, jnp.int32)
        for g in range(4):
            vec = plsc.load_gather(in_v, [dixs[g], ls])
            op_v[rp, pl.ds(qb + g * 16, 16)] = vec


def _detr_body(wt_hbm, tl_hbm, in0, in1, op0, op1, si0, si1, so0, so1):
    wid = lax.axis_index("s") * NC + lax.axis_index("c")
    base = wid * PER_W0
    iota16 = lax.broadcasted_iota(jnp.int32, (16,), 0)
    dixs = [iota16 + g * 16 for g in range(4)]

    ins = (in0, in1)
    ops = (op0, op1)
    si = (si0, si1)
    so = (so0, so1)

    def src(c):
        return wt_hbm.at[:, pl.ds(c * LANE, LANE)]

    def load(c, p):
        return pltpu.make_async_copy(src(c), ins[p].at[:, pl.ds(0, LANE)],
                                     si[p])

    def wb(op_v, c, sem):
        return pltpu.make_async_copy(
            op_v, tl_hbm.at[pl.ds(c * EMBED_DIM, EMBED_DIM)], sem)

    load(base, 0).start()
    load(base + 1, 1).start()

    def pair(it, carry):
        for p in range(2):
            j = 2 * it + p
            c = base + j
            load(c, p).wait()
            @pl.when(j >= 2)
            def _():
                wb(ops[p], c, so[p]).wait()
            _detr_tile(ins[p], ops[p], dixs)
            @pl.when(j + 2 < PER_W0)
            def _():
                load(c + 2, p).start()
            wb(ops[p], c, so[p]).start()
        return carry

    lax.fori_loop(0, PER_W0 // 2, pair, 0)

    for p in range(2):
        wb(ops[p], base + PER_W0 - 2 + p, so[p]).wait()

    @pl.when(wid < NTC0 - NW * PER_W0)
    def _():
        c = NW * PER_W0 + wid
        load(c, 0).start()
        load(c, 0).wait()
        _detr_tile(in0, op0, dixs)
        wb(op0, c, so0).start()
        wb(op0, c, so0).wait()


@jax.jit
def _detr(wt):
    mesh = plsc.VectorSubcoreMesh(core_axis_name="c", subcore_axis_name="s")
    k = pl.kernel(
        _detr_body,
        out_type=jax.ShapeDtypeStruct((VOCAB // 2, LANE), jnp.float32),
        mesh=mesh,
        scratch_types=[
            pltpu.VMEM((EMBED_DIM, 129), jnp.float32),
            pltpu.VMEM((EMBED_DIM, 129), jnp.float32),
            pltpu.VMEM((EMBED_DIM, LANE), jnp.float32),
            pltpu.VMEM((EMBED_DIM, LANE), jnp.float32),
            pltpu.SemaphoreType.DMA,
            pltpu.SemaphoreType.DMA,
            pltpu.SemaphoreType.DMA,
            pltpu.SemaphoreType.DMA,
        ],
        compiler_params=pltpu.CompilerParams(
            use_tc_tiling_on_sc=True, needs_layout_passes=False),
    )
    return k(wt)


def _body(weight_hbm, idx_hbm, out_hbm,
          idx_v, rows0, rows1, cols0, cols1, sg0, sg1, so0, so1):
    wid = lax.axis_index("s") * NC + lax.axis_index("c")
    pltpu.sync_copy(idx_hbm.at[wid], idx_v)
    iota16 = lax.broadcasted_iota(jnp.int32, (16,), 0)
    dixs = [iota16 + g * 16 for g in range(4)]

    rows = (rows0, rows1)
    cols = (cols0, cols1)
    sg = (sg0, sg1)
    so = (so0, so1)

    pltpu.async_copy(weight_hbm.at[idx_v.at[0]], rows0, sg0)
    pltpu.async_copy(weight_hbm.at[idx_v.at[1]], rows1, sg1)

    def pair(it, carry):
        for p in range(2):
            j = 2 * it + p
            t = wid * PER_W + j
            f = t // NBT
            bt = t - f * NBT
            pltpu.make_async_copy(
                weight_hbm.at[idx_v.at[j]], rows[p], sg[p]).wait()
            @pl.when(j >= 2)
            def _():
                for dt in range(8):
                    pltpu.make_async_copy(
                        cols[p].at[pl.ds(dt * 8, 8), pl.ds(0, LANE)],
                        out_hbm.at[f, dt, bt], so[p]).wait()
            _transpose_tile(rows[p], cols[p], dixs)
            @pl.when(j + 2 < PER_W)
            def _():
                pltpu.async_copy(
                    weight_hbm.at[idx_v.at[j + 2]], rows[p], sg[p])
            for dt in range(8):
                pltpu.async_copy(cols[p].at[pl.ds(dt * 8, 8), pl.ds(0, LANE)],
                                 out_hbm.at[f, dt, bt], so[p])
        return carry

    lax.fori_loop(0, PER_W // 2, pair, 0)

    for p in range(2):
        j = PER_W - 2 + p
        t = wid * PER_W + j
        f = t // NBT
        bt = t - f * NBT
        for dt in range(8):
            pltpu.make_async_copy(cols[p].at[pl.ds(dt * 8, 8), pl.ds(0, LANE)],
                                  out_hbm.at[f, dt, bt], so[p]).wait()


@jax.jit
def _embed(idx, weight):
    mesh = plsc.VectorSubcoreMesh(core_axis_name="c", subcore_axis_name="s")
    k = pl.kernel(
        _body,
        out_type=jax.ShapeDtypeStruct((FIELDS, 8, NBT, 8, LANE), jnp.float32),
        mesh=mesh,
        scratch_types=[
            pltpu.VMEM((PER_W, LANE), jnp.int32),
            pltpu.VMEM((LANE, EMBED_DIM), jnp.float32),
            pltpu.VMEM((LANE, EMBED_DIM), jnp.float32),
            pltpu.VMEM((EMBED_DIM, 129), jnp.float32),
            pltpu.VMEM((EMBED_DIM, 129), jnp.float32),
            pltpu.SemaphoreType.DMA,
            pltpu.SemaphoreType.DMA,
            pltpu.SemaphoreType.DMA,
            pltpu.SemaphoreType.DMA,
        ],
        compiler_params=pltpu.CompilerParams(
            use_tc_tiling_on_sc=False, needs_layout_passes=False),
    )
    return k(weight, idx)


def kernel(input, weight):
    idx = input.astype(jnp.int32).T.reshape(NW, PER_W, LANE)
    tl = _detr(weight.T)
    tail = weight[NTC0 * LANE:].reshape(32, LANE)
    tl = lax.dynamic_update_slice(tl, tail, (NTC0 * EMBED_DIM, 0))
    out5 = _embed(idx, tl.reshape(VOCAB, EMBED_DIM))
    return (out5.transpose(2, 4, 0, 1, 3)
            .reshape(BATCH, FIELDS, EMBED_DIM))


# speedup vs baseline: 1.3412x; 1.3412x over previous
import jax
import jax.numpy as jnp
from jax import lax
from jax.experimental import pallas as pl
from jax.experimental.pallas import tpu as pltpu
from jax.experimental.pallas import tpu_sc as plsc

VOCAB = 1000000
EMBED_DIM = 64
BATCH = 16384
FIELDS = 26

NC = 2
NS = 16
NW = NC * NS

NBT = BATCH // 128
NTC = FIELDS * NBT
PER_W = NTC // NW
LANE = 128


def _transpose_tile(rows_v, cols_v, dixs):
    @plsc.parallel_loop(0, LANE, unroll=8)
    def _(l):
        ls = jnp.full((16,), l, jnp.int32)
        for g in range(4):
            vec = rows_v[l, pl.ds(g * 16, 16)]
            plsc.store_scatter(cols_v, [dixs[g], ls], vec)


def _body(weight_hbm, idx_hbm, out_hbm,
          idx_v, rows0, rows1, cols0, cols1, sg0, sg1, so0, so1):
    wid = lax.axis_index("s") * NC + lax.axis_index("c")
    pltpu.sync_copy(idx_hbm.at[wid], idx_v)
    iota16 = lax.broadcasted_iota(jnp.int32, (16,), 0)
    dixs = [iota16 + g * 16 for g in range(4)]

    rows = (rows0, rows1)
    cols = (cols0, cols1)
    sg = (sg0, sg1)
    so = (so0, so1)

    pltpu.async_copy(weight_hbm.at[idx_v.at[0]], rows0, sg0)
    pltpu.async_copy(weight_hbm.at[idx_v.at[1]], rows1, sg1)

    def pair(it, carry):
        for p in range(2):
            j = 2 * it + p
            t = wid * PER_W + j
            f = t // NBT
            bt = t - f * NBT
            pltpu.make_async_copy(
                weight_hbm.at[idx_v.at[j]], rows[p], sg[p]).wait()
            @pl.when(j >= 2)
            def _():
                for dt in range(8):
                    pltpu.make_async_copy(
                        cols[p].at[pl.ds(dt * 8, 8), pl.ds(0, LANE)],
                        out_hbm.at[f, dt, bt], so[p]).wait()
            _transpose_tile(rows[p], cols[p], dixs)
            @pl.when(j + 2 < PER_W)
            def _():
                pltpu.async_copy(
                    weight_hbm.at[idx_v.at[j + 2]], rows[p], sg[p])
            for dt in range(8):
                pltpu.async_copy(cols[p].at[pl.ds(dt * 8, 8), pl.ds(0, LANE)],
                                 out_hbm.at[f, dt, bt], so[p])
        return carry

    lax.fori_loop(0, PER_W // 2, pair, 0)

    for p in range(2):
        j = PER_W - 2 + p
        t = wid * PER_W + j
        f = t // NBT
        bt = t - f * NBT
        for dt in range(8):
            pltpu.make_async_copy(cols[p].at[pl.ds(dt * 8, 8), pl.ds(0, LANE)],
                                  out_hbm.at[f, dt, bt], so[p]).wait()


@jax.jit
def _embed(idx, weight):
    mesh = plsc.VectorSubcoreMesh(core_axis_name="c", subcore_axis_name="s")
    k = pl.kernel(
        _body,
        out_type=jax.ShapeDtypeStruct((FIELDS, 8, NBT, 8, LANE), jnp.float32),
        mesh=mesh,
        scratch_types=[
            pltpu.VMEM((PER_W, LANE), jnp.int32),
            pltpu.VMEM((LANE, LANE), jnp.float32),
            pltpu.VMEM((LANE, LANE), jnp.float32),
            pltpu.VMEM((EMBED_DIM, 129), jnp.float32),
            pltpu.VMEM((EMBED_DIM, 129), jnp.float32),
            pltpu.SemaphoreType.DMA,
            pltpu.SemaphoreType.DMA,
            pltpu.SemaphoreType.DMA,
            pltpu.SemaphoreType.DMA,
        ],
        compiler_params=pltpu.CompilerParams(
            use_tc_tiling_on_sc=False, needs_layout_passes=False),
    )
    return k(weight, idx)


def kernel(input, weight):
    idx = input.astype(jnp.int32).T.reshape(NW, PER_W, LANE)
    wp = jnp.pad(weight, ((0, 0), (0, LANE - EMBED_DIM)))
    out5 = _embed(idx, wp)
    return (out5.transpose(2, 4, 0, 1, 3)
            .reshape(BATCH, FIELDS, EMBED_DIM))
